# Initial kernel scaffold; baseline (speedup 1.0000x reference)
import jax
import jax.numpy as jnp
from jax.experimental import pallas as pl


def kernel(x_0, x_1, x_2, lam_0, U_0, lam_1, U_1, lam_2, U_2, B1, B2, W_in_0, b_in_0, W_in_1, b_in_1, W_in_2, b_in_2, Ws0, Ws1, Ws2, W01, W10, W12, W21):
    raise NotImplementedError("write your pallas kernel here")


# speedup vs baseline: 1.5154x; 1.5154x over previous
import functools

import jax
import jax.numpy as jnp
from jax.experimental import pallas as pl
from jax.experimental.pallas import tpu as pltpu

N0, N1, N2 = 2048, 6144, 4096
D = 128
K = 128
CB1 = 256
SB1 = N1 // CB1
CB2 = 128
SB2 = N2 // CB2

_B2_L0 = SB1
_COMB0 = SB1 + SB2
_B1_L1 = _COMB0 + 1
_B2_L1 = _B1_L1 + SB1
_COMB1 = _B2_L1 + SB2
_STEPS = _COMB1 + 1

_dot = functools.partial(
    jax.lax.dot_general,
    preferred_element_type=jnp.float32,
    precision=jax.lax.Precision.DEFAULT,
)


def _mm(a, b):
    return _dot(a, b, dimension_numbers=(((1,), (0,)), ((), ())))


def _tmm(a, b):
    return _dot(a, b, dimension_numbers=(((0,), (0,)), ((), ())))


def _body(x0, x1, x2, u0, u1, u2, s0, s1, s2, b1m, b2m,
          wi0, bi0, wi1, bi1, wi2, bi2, ws0, ws1, ws2, w01, w10, w12, w21,
          y0, y1, y2,
          h0s, h1s, h2s, a0s, a1s, a2s, a3s):
    i = pl.program_id(0)

    @pl.when(i == 0)
    def _proj():
        h0s[...] = _mm(x0[...], wi0[...]) + bi0[...]
        h1s[...] = _mm(x1[...], wi1[...]) + bi1[...]
        h2s[...] = _mm(x2[...], wi2[...]) + bi2[...]

    in_b1 = (i < SB1) | ((i >= _B1_L1) & (i < _B1_L1 + SB1))

    @pl.when(in_b1)
    def _b1():
        j = jnp.where(i < SB1, i, i - _B1_L1)
        off = pl.multiple_of(j * CB1, CB1)
        blk = b1m[...]

        @pl.when((i == 0) | (i == _B1_L1))
        def _z():
            a0s[...] = jnp.zeros_like(a0s)

        a0s[...] += _mm(blk, h1s[pl.ds(off, CB1), :])
        a1s[pl.ds(off, CB1), :] = _tmm(blk, h0s[...])

    in_b2 = ((i >= _B2_L0) & (i < _COMB0)) | ((i >= _B2_L1) & (i < _COMB1))

    @pl.when(in_b2)
    def _b2():
        j = jnp.where(i < _COMB0, i - _B2_L0, i - _B2_L1)
        off = pl.multiple_of(j * CB2, CB2)
        blk = b2m[...]

        @pl.when((i == _B2_L0) | (i == _B2_L1))
        def _z():
            a2s[...] = jnp.zeros_like(a2s)

        a2s[...] += _mm(blk, h2s[pl.ds(off, CB2), :])
        a3s[pl.ds(off, CB2), :] = _tmm(blk, h1s[...])

    def _spectral(u, hs, sc, ws, l):
        xt = _tmm(u[...], hs[...])
        g = (_mm(xt, ws[l, 0])
             + _mm(sc[0] * xt, ws[l, 1])
             + _mm(sc[1] * xt, ws[l, 2]))
        return _mm(u[...], g)

    def _combine(l, o0, o1, o2):
        r0 = _spectral(u0, h0s, s0, ws0, l) + _mm(a0s[...], w01[l])
        r1 = (_spectral(u1, h1s, s1, ws1, l)
              + _mm(a1s[...], w10[l]) + _mm(a2s[...], w12[l]))
        r2 = _spectral(u2, h2s, s2, ws2, l) + _mm(a3s[...], w21[l])
        o0[...] = r0
        o1[...] = r1
        o2[...] = r2

    @pl.when(i == _COMB0)
    def _c0():
        _combine(0, h0s, h1s, h2s)

    @pl.when(i == _COMB1)
    def _c1():
        _combine(1, y0, y1, y2)


def _full(shape):
    nd = len(shape)
    return pl.BlockSpec(shape, lambda i, _nd=nd: (0,) * _nd)


def _b1_idx(i):
    j = jnp.where(i < _COMB0, i, i - _B1_L1)
    return (0, jnp.clip(j, 0, SB1 - 1))


def _b2_idx(i):
    j = jnp.where(i < _COMB0, i - _B2_L0, i - _B2_L1)
    return (0, jnp.clip(j, 0, SB2 - 1))


def kernel(x_0, x_1, x_2, lam_0, U_0, lam_1, U_1, lam_2, U_2, B1, B2,
           W_in_0, b_in_0, W_in_1, b_in_1, W_in_2, b_in_2,
           Ws0, Ws1, Ws2, W01, W10, W12, W21):
    rr = jnp.array([1.0, 2.0], dtype=jnp.float32)

    def scales(lam):
        s = jnp.exp(-rr[:, None] * lam[None, :])
        return jnp.broadcast_to(s[:, :, None], (2, K, D))

    s0, s1, s2 = scales(lam_0), scales(lam_1), scales(lam_2)
    bi0 = b_in_0.reshape(1, D)
    bi1 = b_in_1.reshape(1, D)
    bi2 = b_in_2.reshape(1, D)

    in_specs = [
        _full((N0, D)), _full((N1, D)), _full((N2, D)),
        _full((N0, K)), _full((N1, K)), _full((N2, K)),
        _full((2, K, D)), _full((2, K, D)), _full((2, K, D)),
        pl.BlockSpec((N0, CB1), _b1_idx),
        pl.BlockSpec((N1, CB2), _b2_idx),
        _full((D, D)), _full((1, D)),
        _full((D, D)), _full((1, D)),
        _full((D, D)), _full((1, D)),
        _full((2, 3, D, D)), _full((2, 3, D, D)), _full((2, 3, D, D)),
        _full((2, D, D)), _full((2, D, D)), _full((2, D, D)), _full((2, D, D)),
    ]
    out_specs = [_full((N0, D)), _full((N1, D)), _full((N2, D))]
    out_shape = [
        jax.ShapeDtypeStruct((N0, D), jnp.float32),
        jax.ShapeDtypeStruct((N1, D), jnp.float32),
        jax.ShapeDtypeStruct((N2, D), jnp.float32),
    ]
    scratch_shapes = [
        pltpu.VMEM((N0, D), jnp.float32),
        pltpu.VMEM((N1, D), jnp.float32),
        pltpu.VMEM((N2, D), jnp.float32),
        pltpu.VMEM((N0, D), jnp.float32),
        pltpu.VMEM((N1, D), jnp.float32),
        pltpu.VMEM((N1, D), jnp.float32),
        pltpu.VMEM((N2, D), jnp.float32),
    ]

    y0, y1, y2 = pl.pallas_call(
        _body,
        grid=(_STEPS,),
        in_specs=in_specs,
        out_specs=out_specs,
        out_shape=out_shape,
        scratch_shapes=scratch_shapes,
        compiler_params=pltpu.CompilerParams(
            dimension_semantics=("arbitrary",),
        ),
    )(x_0, x_1, x_2, U_0, U_1, U_2, s0, s1, s2, B1, B2,
      W_in_0, bi0, W_in_1, bi1, W_in_2, bi2,
      Ws0, Ws1, Ws2, W01, W10, W12, W21)
    return (y0, y1, y2)


# speedup vs baseline: 1.6710x; 1.1027x over previous
import functools

import jax
import jax.numpy as jnp
from jax.experimental import pallas as pl
from jax.experimental.pallas import tpu as pltpu

N0, N1, N2 = 2048, 6144, 4096
D = 128
K = 128
CB1 = 512
SB1 = N1 // CB1
CB2 = 256
SB2 = N2 // CB2

_B2_L0 = SB1
_COMB0 = SB1 + SB2
_B1_L1 = _COMB0 + 1
_B2_L1 = _B1_L1 + SB1
_COMB1 = _B2_L1 + SB2
_STEPS = _COMB1 + 1

_BF = jnp.bfloat16

_dot = functools.partial(
    jax.lax.dot_general,
    preferred_element_type=jnp.float32,
    precision=jax.lax.Precision.DEFAULT,
)


def _mm(a, b):
    return _dot(a, b, dimension_numbers=(((1,), (0,)), ((), ())))


def _tmm(a, b):
    return _dot(a, b, dimension_numbers=(((0,), (0,)), ((), ())))


def _body(x0, x1, x2, u0, u1, u2, s0, s1, s2, b1m, b2m,
          wi0, bi0, wi1, bi1, wi2, bi2, ws0, ws1, ws2, w01, w10, w12, w21,
          y0, y1, y2,
          h0s, h1s, h2s, a0s, a1s, a2s, a3s):
    i = pl.program_id(0)

    @pl.when(i == 0)
    def _proj():
        h0s[...] = (_mm(x0[...], wi0[...]) + bi0[...]).astype(_BF)
        h1s[...] = (_mm(x1[...], wi1[...]) + bi1[...]).astype(_BF)
        h2s[...] = (_mm(x2[...], wi2[...]) + bi2[...]).astype(_BF)

    in_b1 = (i < SB1) | ((i >= _B1_L1) & (i < _B1_L1 + SB1))

    @pl.when(in_b1)
    def _b1():
        j = jnp.where(i < SB1, i, i - _B1_L1)
        off = pl.multiple_of(j * CB1, CB1)
        blk = b1m[...].astype(_BF)

        @pl.when((i == 0) | (i == _B1_L1))
        def _z():
            a0s[...] = jnp.zeros_like(a0s)

        a0s[...] += _mm(blk, h1s[pl.ds(off, CB1), :])
        a1s[pl.ds(off, CB1), :] = _tmm(blk, h0s[...])

    in_b2 = ((i >= _B2_L0) & (i < _COMB0)) | ((i >= _B2_L1) & (i < _COMB1))

    @pl.when(in_b2)
    def _b2():
        j = jnp.where(i < _COMB0, i - _B2_L0, i - _B2_L1)
        off = pl.multiple_of(j * CB2, CB2)
        blk = b2m[...].astype(_BF)

        @pl.when((i == _B2_L0) | (i == _B2_L1))
        def _z():
            a2s[...] = jnp.zeros_like(a2s)

        a2s[...] += _mm(blk, h2s[pl.ds(off, CB2), :])
        a3s[pl.ds(off, CB2), :] = _tmm(blk, h1s[...])

    def _spectral(u, hs, sc, ws, l):
        xt = _tmm(u[...], hs[...])
        g = (_mm(xt.astype(_BF), ws[l, 0])
             + _mm((sc[0] * xt).astype(_BF), ws[l, 1])
             + _mm((sc[1] * xt).astype(_BF), ws[l, 2]))
        return _mm(u[...], g.astype(_BF))

    def _combine(l, o0, o1, o2, out_dtype):
        r0 = (_spectral(u0, h0s, s0, ws0, l)
              + _mm(a0s[...].astype(_BF), w01[l]))
        r1 = (_spectral(u1, h1s, s1, ws1, l)
              + _mm(a1s[...].astype(_BF), w10[l])
              + _mm(a2s[...].astype(_BF), w12[l]))
        r2 = (_spectral(u2, h2s, s2, ws2, l)
              + _mm(a3s[...].astype(_BF), w21[l]))
        o0[...] = r0.astype(out_dtype)
        o1[...] = r1.astype(out_dtype)
        o2[...] = r2.astype(out_dtype)

    @pl.when(i == _COMB0)
    def _c0():
        _combine(0, h0s, h1s, h2s, _BF)

    @pl.when(i == _COMB1)
    def _c1():
        _combine(1, y0, y1, y2, jnp.float32)


def _full(shape):
    nd = len(shape)
    return pl.BlockSpec(shape, lambda i, _nd=nd: (0,) * _nd)


def _b1_idx(i):
    j = jnp.where(i < _COMB0, i, i - _B1_L1)
    return (0, jnp.clip(j, 0, SB1 - 1))


def _b2_idx(i):
    j = jnp.where(i < _COMB0, i - _B2_L0, i - _B2_L1)
    return (0, jnp.clip(j, 0, SB2 - 1))


def kernel(x_0, x_1, x_2, lam_0, U_0, lam_1, U_1, lam_2, U_2, B1, B2,
           W_in_0, b_in_0, W_in_1, b_in_1, W_in_2, b_in_2,
           Ws0, Ws1, Ws2, W01, W10, W12, W21):
    rr = jnp.array([1.0, 2.0], dtype=jnp.float32)

    def scales(lam):
        s = jnp.exp(-rr[:, None] * lam[None, :])
        return jnp.broadcast_to(s[:, :, None], (2, K, D))

    s0, s1, s2 = scales(lam_0), scales(lam_1), scales(lam_2)
    bi0 = b_in_0.reshape(1, D)
    bi1 = b_in_1.reshape(1, D)
    bi2 = b_in_2.reshape(1, D)

    in_specs = [
        _full((N0, D)), _full((N1, D)), _full((N2, D)),
        _full((N0, K)), _full((N1, K)), _full((N2, K)),
        _full((2, K, D)), _full((2, K, D)), _full((2, K, D)),
        pl.BlockSpec((N0, CB1), _b1_idx),
        pl.BlockSpec((N1, CB2), _b2_idx),
        _full((D, D)), _full((1, D)),
        _full((D, D)), _full((1, D)),
        _full((D, D)), _full((1, D)),
        _full((2, 3, D, D)), _full((2, 3, D, D)), _full((2, 3, D, D)),
        _full((2, D, D)), _full((2, D, D)), _full((2, D, D)), _full((2, D, D)),
    ]
    out_specs = [_full((N0, D)), _full((N1, D)), _full((N2, D))]
    out_shape = [
        jax.ShapeDtypeStruct((N0, D), jnp.float32),
        jax.ShapeDtypeStruct((N1, D), jnp.float32),
        jax.ShapeDtypeStruct((N2, D), jnp.float32),
    ]
    scratch_shapes = [
        pltpu.VMEM((N0, D), _BF),
        pltpu.VMEM((N1, D), _BF),
        pltpu.VMEM((N2, D), _BF),
        pltpu.VMEM((N0, D), jnp.float32),
        pltpu.VMEM((N1, D), jnp.float32),
        pltpu.VMEM((N1, D), jnp.float32),
        pltpu.VMEM((N2, D), jnp.float32),
    ]

    y0, y1, y2 = pl.pallas_call(
        _body,
        grid=(_STEPS,),
        in_specs=in_specs,
        out_specs=out_specs,
        out_shape=out_shape,
        scratch_shapes=scratch_shapes,
        compiler_params=pltpu.CompilerParams(
            dimension_semantics=("arbitrary",),
        ),
    )(x_0.astype(_BF), x_1.astype(_BF), x_2.astype(_BF),
      U_0.astype(_BF), U_1.astype(_BF), U_2.astype(_BF), s0, s1, s2, B1, B2,
      W_in_0.astype(_BF), bi0, W_in_1.astype(_BF), bi1, W_in_2.astype(_BF),
      bi2, Ws0.astype(_BF), Ws1.astype(_BF), Ws2.astype(_BF),
      W01.astype(_BF), W10.astype(_BF), W12.astype(_BF), W21.astype(_BF))
    return (y0, y1, y2)
